# baseline (speedup 1.0000x reference)
import functools
import math

import jax
import jax.numpy as jnp
from jax import lax
from jax.experimental import pallas as pl
from jax.experimental.pallas import tpu as pltpu
from jax.experimental.pallas import tpu_sc as plsc

_NC = 2
_NS = 16
_LANES = 16
_NW = _NC * _NS

_CH = 128


@functools.partial(jax.jit, static_argnames=("b_total", "d_model"))
def _emb_lookup(x_flat, table, *, b_total, d_model):
    b_per_w = b_total // _NW
    n_chunks = b_per_w // _CH
    scale = jnp.float32(math.sqrt(float(d_model)))
    vecs_per_row = d_model // _LANES

    mesh = plsc.VectorSubcoreMesh(core_axis_name="c", subcore_axis_name="s")

    @functools.partial(
        pl.kernel,
        mesh=mesh,
        out_type=jax.ShapeDtypeStruct((b_total, d_model), jnp.float32),
        scratch_types=[
            pltpu.VMEM((b_per_w,), jnp.int32),
            pltpu.VMEM((_CH, d_model), jnp.float32),
            pltpu.VMEM((_CH, d_model), jnp.float32),
            pltpu.SemaphoreType.DMA,
            pltpu.SemaphoreType.DMA,
            pltpu.SemaphoreType.DMA,
            pltpu.SemaphoreType.DMA,
        ],
    )
    def body(idx_hbm, table_hbm, out_hbm, idx_v, buf0, buf1, g0, g1, s0, s1):
        wid = lax.axis_index("s") * _NC + lax.axis_index("c")
        base = wid * b_per_w
        pltpu.sync_copy(idx_hbm.at[pl.ds(base, b_per_w)], idx_v)

        bufs = (buf0, buf1)
        gsems = (g0, g1)
        ssems = (s0, s1)

        def gather_desc(c):
            b = c % 2
            return pltpu.make_async_copy(
                table_hbm.at[idx_v.at[pl.ds(c * _CH, _CH)]], bufs[b], gsems[b]
            )

        def store_desc(c):
            b = c % 2
            return pltpu.make_async_copy(
                bufs[b], out_hbm.at[pl.ds(base + c * _CH, _CH)], ssems[b]
            )

        def scale_chunk(b):
            buf = bufs[b]

            def row(r, carry):
                for j in range(vecs_per_row):
                    sl = (r, pl.ds(j * _LANES, _LANES))
                    buf[sl] = buf[sl] * scale
                return carry

            lax.fori_loop(0, _CH, row, 0, unroll=2)

        gather_desc(0).start()
        for c in range(n_chunks):
            if c + 1 < n_chunks:
                if c >= 1:
                    store_desc(c - 1).wait()
                gather_desc(c + 1).start()
            gather_desc(c).wait()
            scale_chunk(c % 2)
            store_desc(c).start()
        store_desc(n_chunks - 1).wait()
        if n_chunks >= 2:
            store_desc(n_chunks - 2).wait()

    return body(x_flat, table)


def kernel(x, table):
    n0, n1 = x.shape
    d_model = table.shape[1]
    n1p = (n1 + 7) // 8 * 8
    xp = jnp.pad(x.astype(jnp.int32), ((0, 0), (0, n1p - n1)))
    b_total = n0 * n1p
    out = _emb_lookup(xp.reshape(b_total), table, b_total=b_total, d_model=d_model)
    return out.reshape(n0, n1p, d_model)[:, :n1, :]


# speedup vs baseline: 6.2191x; 6.2191x over previous
import functools
import math

import jax
import jax.numpy as jnp
from jax import lax
from jax.experimental import pallas as pl
from jax.experimental.pallas import tpu as pltpu
from jax.experimental.pallas import tpu_sc as plsc

_NC = 2
_NS = 16
_LANES = 16
_NW = _NC * _NS

_CH = 128


@functools.partial(jax.jit, static_argnames=("b_total", "d_model"))
def _emb_lookup(x_flat, table, *, b_total, d_model):
    b_per_w = b_total // _NW
    n_chunks = b_per_w // _CH
    scale = jnp.float32(math.sqrt(float(d_model)))
    vecs_per_row = d_model // _LANES

    mesh = plsc.VectorSubcoreMesh(core_axis_name="c", subcore_axis_name="s")

    @functools.partial(
        pl.kernel,
        mesh=mesh,
        out_type=jax.ShapeDtypeStruct((b_total, d_model), jnp.float32),
        scratch_types=[
            pltpu.VMEM((b_per_w,), jnp.int32),
            pltpu.VMEM((_CH, d_model), jnp.float32),
            pltpu.VMEM((_CH, d_model), jnp.float32),
            pltpu.SemaphoreType.DMA,
            pltpu.SemaphoreType.DMA,
            pltpu.SemaphoreType.DMA,
            pltpu.SemaphoreType.DMA,
        ],
    )
    def body(idx_hbm, table_hbm, out_hbm, idx_v, buf0, buf1, g0, g1, s0, s1):
        wid = lax.axis_index("s") * _NC + lax.axis_index("c")
        base = wid * b_per_w
        pltpu.sync_copy(idx_hbm.at[pl.ds(base, b_per_w)], idx_v)

        bufs = (buf0, buf1)
        gsems = (g0, g1)
        ssems = (s0, s1)

        def gather_desc(c):
            b = c % 2
            return pltpu.make_async_copy(
                table_hbm.at[idx_v.at[pl.ds(c * _CH, _CH)]], bufs[b], gsems[b]
            )

        def store_desc(c):
            b = c % 2
            return pltpu.make_async_copy(
                bufs[b], out_hbm.at[pl.ds(base + c * _CH, _CH)], ssems[b]
            )

        def scale_chunk(b):
            buf = bufs[b]

            def row(r, carry):
                for j in range(vecs_per_row):
                    sl = (r, pl.ds(j * _LANES, _LANES))
                    buf[sl] = buf[sl] * scale
                return carry

            lax.fori_loop(0, _CH, row, 0, unroll=2)

        gather_desc(0).start()
        for c in range(n_chunks):
            if c + 1 < n_chunks:
                if c >= 1:
                    store_desc(c - 1).wait()
                gather_desc(c + 1).start()
            gather_desc(c).wait()
            scale_chunk(c % 2)
            store_desc(c).start()
        store_desc(n_chunks - 1).wait()
        if n_chunks >= 2:
            store_desc(n_chunks - 2).wait()

    return body(x_flat, table)


def kernel(x, table):
    n0, n1 = x.shape
    d_model = table.shape[1]
    n1p = (n1 + 7) // 8 * 8
    npad = n1p - n1
    vocab = table.shape[0]
    pad_idx = (jnp.arange(n0 * npad, dtype=jnp.int32) % vocab).reshape(n0, npad)
    xp = jnp.concatenate([x.astype(jnp.int32), pad_idx], axis=1)
    b_total = n0 * n1p
    out = _emb_lookup(xp.reshape(b_total), table, b_total=b_total, d_model=d_model)
    return out.reshape(n0, n1p, d_model)[:, :n1, :]


# speedup vs baseline: 7.6688x; 1.2331x over previous
import functools
import math

import jax
import jax.numpy as jnp
from jax import lax
from jax.experimental import pallas as pl
from jax.experimental.pallas import tpu as pltpu
from jax.experimental.pallas import tpu_sc as plsc

_NC = 2
_NS = 16
_LANES = 16
_NW = _NC * _NS

_NBUF = 8
_LOOKAHEAD = 4


@functools.partial(jax.jit, static_argnames=("n1",))
def _emb_lookup(xp, table, *, n1):
    n0, n1p = xp.shape
    d_model = table.shape[1]
    g_per_w = n0 // _NW
    scale = jnp.float32(math.sqrt(float(d_model)))
    vecs_per_row = d_model // _LANES

    mesh = plsc.VectorSubcoreMesh(core_axis_name="c", subcore_axis_name="s")

    @functools.partial(
        pl.kernel,
        mesh=mesh,
        out_type=jax.ShapeDtypeStruct((n0, n1, d_model), jnp.float32),
        scratch_types=[
            pltpu.VMEM((g_per_w, n1p), jnp.int32),
            [pltpu.VMEM((n1p, d_model), jnp.float32) for _ in range(_NBUF)],
            [pltpu.SemaphoreType.DMA for _ in range(_NBUF)],
            [pltpu.SemaphoreType.DMA for _ in range(_NBUF)],
        ],
        compiler_params=pltpu.CompilerParams(use_tc_tiling_on_sc=True),
    )
    def body(xp_hbm, table_hbm, out_hbm, idx_v, bufs, gsems, ssems):
        wid = lax.axis_index("s") * _NC + lax.axis_index("c")
        gbase = wid * g_per_w
        pltpu.sync_copy(xp_hbm.at[pl.ds(gbase, g_per_w), :], idx_v)

        def gather(g, b):
            return pltpu.make_async_copy(
                table_hbm.at[idx_v.at[g]], bufs[b], gsems[b]
            )

        def store(g, b):
            return pltpu.make_async_copy(
                bufs[b].at[pl.ds(0, n1)], out_hbm.at[gbase + g], ssems[b]
            )

        def scale_buf(b):
            buf = bufs[b]

            def row(r, carry):
                for j in range(vecs_per_row):
                    sl = (r, pl.ds(j * _LANES, _LANES))
                    buf[sl] = buf[sl] * scale
                return carry

            lax.fori_loop(0, n1, row, 0, unroll=2)

        K = _LOOKAHEAD
        G = g_per_w
        for g in range(K):
            gather(g, g % _NBUF).start()
        for g in range(K):
            gather(g + K, (g + K) % _NBUF).start()
            gather(g, g % _NBUF).wait()
            scale_buf(g % _NBUF)
            store(g, g % _NBUF).start()
        steady = (G - 2 * K) // _NBUF * _NBUF

        def outer(io, carry):
            for j in range(_NBUF):
                g = K + io * _NBUF + j
                b_next = (2 * K + j) % _NBUF
                b = (K + j) % _NBUF
                store(g - K, b_next).wait()
                gather(g + K, b_next).start()
                gather(g, b).wait()
                scale_buf(b)
                store(g, b).start()
            return carry

        lax.fori_loop(0, steady // _NBUF, outer, 0)
        for g in range(K + steady, G):
            if g + K < G:
                store(g - K, (g + K) % _NBUF).wait()
                gather(g + K, (g + K) % _NBUF).start()
            gather(g, g % _NBUF).wait()
            scale_buf(g % _NBUF)
            store(g, g % _NBUF).start()
        for g in range(G - 2 * K, G):
            store(g, g % _NBUF).wait()

    return body(xp, table)


def kernel(x, table):
    n0, n1 = x.shape
    d_model = table.shape[1]
    n1p = (n1 + 7) // 8 * 8
    npad = n1p - n1
    vocab = table.shape[0]
    pad_idx = (jnp.arange(n0 * npad, dtype=jnp.int32) % vocab).reshape(n0, npad)
    xp = jnp.concatenate([x.astype(jnp.int32), pad_idx], axis=1)
    return _emb_lookup(xp, table, n1=n1)


# speedup vs baseline: 13.5701x; 1.7695x over previous
import functools
import math

import jax
import jax.numpy as jnp
from jax import lax
from jax.experimental import pallas as pl
from jax.experimental.pallas import tpu as pltpu
from jax.experimental.pallas import tpu_sc as plsc

_NC = 2
_NS = 16
_LANES = 16
_NW = _NC * _NS

_CH = 128
_NBUF = 6
_K = 3


@jax.jit
def _emb_lookup(x_flat, table):
    b_total = x_flat.shape[0]
    d_model = table.shape[1]
    b_per_w = b_total // _NW
    n_chunks = b_per_w // _CH
    scale = jnp.float32(math.sqrt(float(d_model)))
    vecs_per_row = d_model // _LANES

    mesh = plsc.VectorSubcoreMesh(core_axis_name="c", subcore_axis_name="s")

    @functools.partial(
        pl.kernel,
        mesh=mesh,
        out_type=jax.ShapeDtypeStruct((b_total, d_model), jnp.float32),
        scratch_types=[
            pltpu.VMEM((b_per_w,), jnp.int32),
            [pltpu.VMEM((_CH, d_model), jnp.float32) for _ in range(_NBUF)],
            [pltpu.SemaphoreType.DMA for _ in range(_NBUF)],
            [pltpu.SemaphoreType.DMA for _ in range(_NBUF)],
        ],
    )
    def body(idx_hbm, table_hbm, out_hbm, idx_v, bufs, gsems, ssems):
        wid = lax.axis_index("s") * _NC + lax.axis_index("c")
        base = wid * b_per_w
        pltpu.sync_copy(idx_hbm.at[pl.ds(base, b_per_w)], idx_v)

        def gather(c, b):
            return pltpu.make_async_copy(
                table_hbm.at[idx_v.at[pl.ds(c * _CH, _CH)]], bufs[b], gsems[b]
            )

        def store(c, b):
            return pltpu.make_async_copy(
                bufs[b], out_hbm.at[pl.ds(base + c * _CH, _CH)], ssems[b]
            )

        def scale_buf(b):
            buf = bufs[b]

            def row(r, carry):
                for j in range(vecs_per_row):
                    sl = (r, pl.ds(j * _LANES, _LANES))
                    buf[sl] = buf[sl] * scale
                return carry

            lax.fori_loop(0, _CH, row, 0, unroll=2)

        G = n_chunks
        for c in range(_K):
            gather(c, c % _NBUF).start()
        for c in range(_K):
            gather(c + _K, (c + _K) % _NBUF).start()
            gather(c, c % _NBUF).wait()
            scale_buf(c % _NBUF)
            store(c, c % _NBUF).start()
        steady = (G - 2 * _K) // _NBUF * _NBUF

        def outer(io, carry):
            for j in range(_NBUF):
                c = _K + io * _NBUF + j
                b_next = (2 * _K + j) % _NBUF
                b = (_K + j) % _NBUF
                store(c - _K, b_next).wait()
                gather(c + _K, b_next).start()
                gather(c, b).wait()
                scale_buf(b)
                store(c, b).start()
            return carry

        lax.fori_loop(0, steady // _NBUF, outer, 0)
        for c in range(_K + steady, G):
            if c + _K < G:
                store(c - _K, (c + _K) % _NBUF).wait()
                gather(c + _K, (c + _K) % _NBUF).start()
            gather(c, c % _NBUF).wait()
            scale_buf(c % _NBUF)
            store(c, c % _NBUF).start()
        for c in range(G - 2 * _K, G):
            store(c, c % _NBUF).wait()

    return body(x_flat, table)


def kernel(x, table):
    n0, n1 = x.shape
    d_model = table.shape[1]
    x_flat = x.astype(jnp.int32).T.reshape(n0 * n1)
    out = _emb_lookup(x_flat, table)
    return out.reshape(n1, n0, d_model).transpose(1, 0, 2)
